# Initial kernel scaffold; baseline (speedup 1.0000x reference)
import jax
import jax.numpy as jnp
from jax.experimental import pallas as pl


def kernel(pred, prob_map, prob_mask):
    raise NotImplementedError("write your pallas kernel here")


# speedup vs baseline: 11.4665x; 11.4665x over previous
import functools

import jax
import jax.numpy as jnp
from jax import lax
from jax.experimental import pallas as pl
from jax.experimental.pallas import tpu as pltpu
from jax.experimental.pallas import tpu_sc as plsc

N = 2048 * 2048
BINS = 1024
L = 16

_info = plsc.get_sparse_core_info()
NC, NS = _info.num_cores, _info.num_subcores
NW = NC * NS
PER_W = N // NW
CHUNK = 8192
NCH = PER_W // CHUNK


def _make_sc_pass(shift, mask_shift, with_pos):
    mesh = plsc.VectorSubcoreMesh(core_axis_name="c", subcore_axis_name="s")
    out_type = [jax.ShapeDtypeStruct((NW * BINS,), jnp.float32)]
    if with_pos:
        out_type.append(jax.ShapeDtypeStruct((NW * L,), jnp.float32))
    scratch = [
        pltpu.VMEM((BINS * L,), jnp.float32),
        pltpu.VMEM((CHUNK,), jnp.float32),
        pltpu.VMEM((CHUNK,), jnp.float32),
        pltpu.VMEM((BINS,), jnp.float32),
        pltpu.VMEM((L,), jnp.float32),
        pltpu.VMEM((L,), jnp.int32),
    ]

    def body(pred_hbm, prob_hbm, mval_hbm, *rest):
        if with_pos:
            hist_out, pos_out = rest[0], rest[1]
            hist_v, pbuf, mbuf, merged, posbuf, mvalbuf = rest[2:]
        else:
            hist_out = rest[0]
            pos_out = None
            hist_v, pbuf, mbuf, merged, posbuf, mvalbuf = rest[1:]

        wid = lax.axis_index("s") * NC + lax.axis_index("c")
        base = wid * PER_W
        pltpu.sync_copy(mval_hbm, mvalbuf)
        mval = mvalbuf[...]
        lane = lax.iota(jnp.int32, L)
        shv = jnp.full((L,), shift, jnp.int32)
        mshv = jnp.full((L,), mask_shift, jnp.int32)
        binmask = jnp.full((L,), BINS - 1, jnp.int32)
        lsplat = jnp.full((L,), BINS, jnp.int32)
        z16f = jnp.zeros((L,), jnp.float32)
        one16f = jnp.full((L,), 1.0, jnp.float32)

        def zbody(j, c):
            hist_v[pl.ds(j * L, L)] = z16f
            return c

        lax.fori_loop(0, BINS * L // L, zbody, 0)

        def chunk_body(ch, pos_carry):
            off = base + ch * CHUNK
            pltpu.sync_copy(pred_hbm.at[pl.ds(off, CHUNK)], pbuf)
            pltpu.sync_copy(prob_hbm.at[pl.ds(off, CHUNK)], mbuf)

            def vec_body(i, pc):
                p = pbuf[pl.ds(i * L, L)]
                m = mbuf[pl.ds(i * L, L)]
                bits = lax.bitcast_convert_type(p, jnp.int32)
                bin_ = lax.shift_right_logical(bits, shv) & binmask
                keep = lax.shift_right_logical(bits, mshv) == mval
                val = jnp.where(keep, one16f - m, z16f)
                idx = lane * lsplat + bin_
                plsc.addupdate_scatter(hist_v, [idx], val)
                return pc + m

            return lax.fori_loop(0, CHUNK // L, vec_body, pos_carry)

        posacc = lax.fori_loop(0, NCH, chunk_body, z16f)

        def mbody(j, c):
            acc = hist_v[pl.ds(j * L, L)]
            for l in range(1, L):
                acc = acc + hist_v[pl.ds(l * BINS + j * L, L)]
            merged[pl.ds(j * L, L)] = acc
            return c

        lax.fori_loop(0, BINS // L, mbody, 0)
        pltpu.sync_copy(merged, hist_out.at[pl.ds(wid * BINS, BINS)])
        if with_pos:
            posbuf[...] = posacc
            pltpu.sync_copy(posbuf, pos_out.at[pl.ds(wid * L, L)])

    return functools.partial(
        pl.kernel, mesh=mesh, out_type=out_type, scratch_types=scratch,
        compiler_params=pltpu.CompilerParams(needs_layout_passes=False),
    )(body)


_sc_pass1 = _make_sc_pass(20, 30, True)
_sc_pass2 = _make_sc_pass(10, 20, False)
_sc_pass3 = _make_sc_pass(0, 10, False)

TC_ROWS = 4096
TC_COLS = N // TC_ROWS
TC_BLK = 256


def _tc_body(t_ref, pred_ref, prob_ref, pos_out, sel_out):
    i = pl.program_id(0)
    p = pred_ref[...]
    m = prob_ref[...]
    logp = jnp.maximum(jnp.log(p), -100.0)
    log1mp = jnp.maximum(jnp.log(1.0 - p), -100.0)
    pos_part = jnp.sum(m * (-logp))
    negl = (1.0 - m) * (-log1mp)
    bits = lax.bitcast_convert_type(p, jnp.int32)
    sel_part = jnp.sum(jnp.where(bits > t_ref[0], negl, 0.0))

    @pl.when(i == 0)
    def _():
        pos_out[...] = jnp.zeros((1, 1), jnp.float32)
        sel_out[...] = jnp.zeros((1, 1), jnp.float32)

    pos_out[...] += jnp.full((1, 1), pos_part, jnp.float32)
    sel_out[...] += jnp.full((1, 1), sel_part, jnp.float32)


def _tc_sums(t_bits, pred2d, prob2d):
    return pl.pallas_call(
        _tc_body,
        grid=(TC_ROWS // TC_BLK,),
        in_specs=[
            pl.BlockSpec(memory_space=pltpu.SMEM),
            pl.BlockSpec((TC_BLK, TC_COLS), lambda i: (i, 0)),
            pl.BlockSpec((TC_BLK, TC_COLS), lambda i: (i, 0)),
        ],
        out_specs=[
            pl.BlockSpec((1, 1), lambda i: (0, 0)),
            pl.BlockSpec((1, 1), lambda i: (0, 0)),
        ],
        out_shape=[
            jax.ShapeDtypeStruct((1, 1), jnp.float32),
            jax.ShapeDtypeStruct((1, 1), jnp.float32),
        ],
    )(t_bits, pred2d, prob2d)


def _select_bin(hist, want):
    frev = jnp.cumsum(hist[::-1])[::-1]
    b = jnp.sum((frev >= want).astype(jnp.int32)) - 1
    above = jnp.concatenate([frev, jnp.zeros((1,), jnp.float32)])[b + 1]
    return b, above, want - above


def kernel(pred, prob_map, prob_mask):
    predf = pred.reshape(-1)
    probf = prob_map.reshape(-1)

    z16 = jnp.zeros((L,), jnp.int32)
    hist1_rows, pos_rows = _sc_pass1(predf, probf, z16)
    hist1 = hist1_rows.reshape(NW, BINS).sum(axis=0)
    pos_num = pos_rows.sum()
    neg_count = jnp.float32(N) - pos_num
    neg_num = jnp.minimum(neg_count, jnp.floor(pos_num * 3.0))

    b1, above1, r1 = _select_bin(hist1, neg_num)

    mv2 = jnp.full((L,), b1, jnp.int32)
    (hist2_rows,) = _sc_pass2(predf, probf, mv2)
    hist2 = hist2_rows.reshape(NW, BINS).sum(axis=0)
    b2, above2, r2 = _select_bin(hist2, r1)

    mv3 = jnp.full((L,), b1 * BINS + b2, jnp.int32)
    (hist3_rows,) = _sc_pass3(predf, probf, mv3)
    hist3 = hist3_rows.reshape(NW, BINS).sum(axis=0)
    b3, above3, r_final = _select_bin(hist3, r2)

    t_bits = (b1 * (1 << 20) + b2 * (1 << 10) + b3).astype(jnp.int32)

    pred2d = predf.reshape(TC_ROWS, TC_COLS)
    prob2d = probf.reshape(TC_ROWS, TC_COLS)
    pos_sum, sel_sum = _tc_sums(t_bits.reshape(1), pred2d, prob2d)

    t_f = lax.bitcast_convert_type(t_bits, jnp.float32)
    loss_t = -jnp.maximum(jnp.log(1.0 - t_f), -100.0)
    tie_sum = jnp.where(r_final > 0.0, r_final * loss_t, 0.0)

    bce = (pos_sum[0, 0] + sel_sum[0, 0] + tie_sum) / (pos_num + neg_num + 1e-6)
    return bce


# speedup vs baseline: 16.0646x; 1.4010x over previous
import functools

import jax
import jax.numpy as jnp
from jax import lax
from jax.experimental import pallas as pl
from jax.experimental.pallas import tpu as pltpu
from jax.experimental.pallas import tpu_sc as plsc

N = 2048 * 2048
DIM = 2048
BINS = 1024
L = 16

_info = plsc.get_sparse_core_info()
NC, NS = _info.num_cores, _info.num_subcores
NW = NC * NS
ROWS_W = DIM // NW
CH_ROWS = 8
NCH = ROWS_W // CH_ROWS
VECS = CH_ROWS * DIM // L
UNROLL = 4


def _make_sc_pass(shift, mask_shift, with_pos):
    mesh = plsc.VectorSubcoreMesh(core_axis_name="c", subcore_axis_name="s")
    out_type = [jax.ShapeDtypeStruct((NW * BINS,), jnp.float32)]
    if with_pos:
        out_type.append(jax.ShapeDtypeStruct((NW * L,), jnp.float32))
    scratch = [
        pltpu.VMEM((BINS * L,), jnp.float32),
        pltpu.VMEM((CH_ROWS, DIM), jnp.float32),
        pltpu.VMEM((CH_ROWS, DIM), jnp.float32),
        pltpu.VMEM((CH_ROWS, DIM), jnp.float32),
        pltpu.VMEM((CH_ROWS, DIM), jnp.float32),
        pltpu.VMEM((BINS,), jnp.float32),
        pltpu.VMEM((L,), jnp.float32),
        pltpu.VMEM((L,), jnp.int32),
        pltpu.SemaphoreType.DMA,
        pltpu.SemaphoreType.DMA,
    ]

    def body(pred_hbm, prob_hbm, mval_hbm, *rest):
        if with_pos:
            hist_out, pos_out = rest[0], rest[1]
            rest = rest[2:]
        else:
            hist_out = rest[0]
            pos_out = None
            rest = rest[1:]
        hist_v, pb0, pb1, mb0, mb1, merged, posbuf, mvalbuf, sem0, sem1 = rest
        pbufs, mbufs, sems = (pb0, pb1), (mb0, mb1), (sem0, sem1)

        wid = lax.axis_index("s") * NC + lax.axis_index("c")
        row0 = wid * ROWS_W
        pltpu.sync_copy(mval_hbm, mvalbuf)
        mval = mvalbuf[...]
        lane = lax.iota(jnp.int32, L)
        laneB = lane * jnp.full((L,), BINS, jnp.int32)
        shv = jnp.full((L,), shift, jnp.int32)
        mshv = jnp.full((L,), mask_shift, jnp.int32)
        binmask = jnp.full((L,), BINS - 1, jnp.int32)
        z16f = jnp.zeros((L,), jnp.float32)
        one16f = jnp.full((L,), 1.0, jnp.float32)

        def zbody(j, c):
            hist_v[pl.ds(j * L, L)] = z16f
            return c

        lax.fori_loop(0, BINS * L // L, zbody, 0)

        def issue(b, ch):
            r = row0 + ch * CH_ROWS
            pltpu.async_copy(pred_hbm.at[pl.ds(r, CH_ROWS)], pbufs[b], sems[b])
            pltpu.async_copy(prob_hbm.at[pl.ds(r, CH_ROWS)], mbufs[b], sems[b])

        def wait(b):
            src = pred_hbm.at[pl.ds(row0, CH_ROWS)]
            pltpu.make_async_copy(src, pbufs[b], sems[b]).wait()
            pltpu.make_async_copy(src, mbufs[b], sems[b]).wait()

        def process(b, ch, pos_carry):
            wait(b)
            pbuf, mbuf = pbufs[b], mbufs[b]

            def vec_body(j, accs):
                accs = list(accs)
                for r in range(CH_ROWS):
                    for u in range(UNROLL):
                        sl = pl.ds(j * (UNROLL * L) + u * L, L)
                        p = pbuf[r, sl]
                        m = mbuf[r, sl]
                        bits = lax.bitcast_convert_type(p, jnp.int32)
                        bin_ = lax.shift_right_logical(bits, shv) & binmask
                        keep = lax.shift_right_logical(bits, mshv) == mval
                        val = jnp.where(keep, one16f - m, z16f)
                        plsc.addupdate_scatter(hist_v, [laneB + bin_], val)
                        if with_pos:
                            accs[u] = accs[u] + m
                return tuple(accs)

            nj = DIM // (UNROLL * L)
            accs = lax.fori_loop(0, nj, vec_body, pos_carry)
            @pl.when(ch + 2 < NCH)
            def _():
                issue(b, ch + 2)
            return accs

        issue(0, 0)
        issue(1, 1)

        def pair(cp, accs):
            accs = process(0, cp * 2, accs)
            accs = process(1, cp * 2 + 1, accs)
            return accs

        accs = lax.fori_loop(0, NCH // 2, pair, (z16f,) * UNROLL)
        posacc = accs[0]
        for u in range(1, UNROLL):
            posacc = posacc + accs[u]

        def mbody(j, c):
            acc = hist_v[pl.ds(j * L, L)]
            for l in range(1, L):
                acc = acc + hist_v[pl.ds(l * BINS + j * L, L)]
            merged[pl.ds(j * L, L)] = acc
            return c

        lax.fori_loop(0, BINS // L, mbody, 0)
        pltpu.sync_copy(merged, hist_out.at[pl.ds(wid * BINS, BINS)])
        if with_pos:
            posbuf[...] = posacc
            pltpu.sync_copy(posbuf, pos_out.at[pl.ds(wid * L, L)])

    return functools.partial(
        pl.kernel, mesh=mesh, out_type=out_type, scratch_types=scratch,
        compiler_params=pltpu.CompilerParams(
            needs_layout_passes=False, use_tc_tiling_on_sc=True),
    )(body)


_sc_pass1 = _make_sc_pass(20, 30, True)
_sc_pass2 = _make_sc_pass(10, 20, False)
_sc_pass3 = _make_sc_pass(0, 10, False)

TC_ROWS = DIM
TC_COLS = DIM
TC_BLK = 256


def _tc_body(t_ref, pred_ref, prob_ref, pos_out, sel_out):
    i = pl.program_id(0)
    p = pred_ref[...]
    m = prob_ref[...]
    logp = jnp.maximum(jnp.log(p), -100.0)
    log1mp = jnp.maximum(jnp.log(1.0 - p), -100.0)
    pos_part = jnp.sum(m * (-logp))
    negl = (1.0 - m) * (-log1mp)
    bits = lax.bitcast_convert_type(p, jnp.int32)
    sel_part = jnp.sum(jnp.where(bits > t_ref[0], negl, 0.0))

    @pl.when(i == 0)
    def _():
        pos_out[...] = jnp.zeros((1, 1), jnp.float32)
        sel_out[...] = jnp.zeros((1, 1), jnp.float32)

    pos_out[...] += jnp.full((1, 1), pos_part, jnp.float32)
    sel_out[...] += jnp.full((1, 1), sel_part, jnp.float32)


def _tc_sums(t_bits, pred2d, prob2d):
    return pl.pallas_call(
        _tc_body,
        grid=(TC_ROWS // TC_BLK,),
        in_specs=[
            pl.BlockSpec(memory_space=pltpu.SMEM),
            pl.BlockSpec((TC_BLK, TC_COLS), lambda i: (i, 0)),
            pl.BlockSpec((TC_BLK, TC_COLS), lambda i: (i, 0)),
        ],
        out_specs=[
            pl.BlockSpec((1, 1), lambda i: (0, 0)),
            pl.BlockSpec((1, 1), lambda i: (0, 0)),
        ],
        out_shape=[
            jax.ShapeDtypeStruct((1, 1), jnp.float32),
            jax.ShapeDtypeStruct((1, 1), jnp.float32),
        ],
    )(t_bits, pred2d, prob2d)


def _select_bin(hist, want):
    frev = jnp.cumsum(hist[::-1])[::-1]
    b = jnp.sum((frev >= want).astype(jnp.int32)) - 1
    above = jnp.concatenate([frev, jnp.zeros((1,), jnp.float32)])[b + 1]
    return b, above, want - above


def kernel(pred, prob_map, prob_mask):
    predf = pred.reshape(DIM, DIM)
    probf = prob_map.reshape(DIM, DIM)

    z16 = jnp.zeros((L,), jnp.int32)
    hist1_rows, pos_rows = _sc_pass1(predf, probf, z16)
    hist1 = hist1_rows.reshape(NW, BINS).sum(axis=0)
    pos_num = pos_rows.sum()
    neg_count = jnp.float32(N) - pos_num
    neg_num = jnp.minimum(neg_count, jnp.floor(pos_num * 3.0))

    b1, above1, r1 = _select_bin(hist1, neg_num)

    mv2 = jnp.full((L,), b1, jnp.int32)
    (hist2_rows,) = _sc_pass2(predf, probf, mv2)
    hist2 = hist2_rows.reshape(NW, BINS).sum(axis=0)
    b2, above2, r2 = _select_bin(hist2, r1)

    mv3 = jnp.full((L,), b1 * BINS + b2, jnp.int32)
    (hist3_rows,) = _sc_pass3(predf, probf, mv3)
    hist3 = hist3_rows.reshape(NW, BINS).sum(axis=0)
    b3, above3, r_final = _select_bin(hist3, r2)

    t_bits = (b1 * (1 << 20) + b2 * (1 << 10) + b3).astype(jnp.int32)

    pos_sum, sel_sum = _tc_sums(t_bits.reshape(1), predf, probf)

    t_f = lax.bitcast_convert_type(t_bits, jnp.float32)
    loss_t = -jnp.maximum(jnp.log(1.0 - t_f), -100.0)
    tie_sum = jnp.where(r_final > 0.0, r_final * loss_t, 0.0)

    bce = (pos_sum[0, 0] + sel_sum[0, 0] + tie_sum) / (pos_num + neg_num + 1e-6)
    return bce


# speedup vs baseline: 33.7317x; 2.0998x over previous
import functools

import jax
import jax.numpy as jnp
from jax import lax
from jax.experimental import pallas as pl
from jax.experimental.pallas import tpu as pltpu
from jax.experimental.pallas import tpu_sc as plsc

N = 2048 * 2048
DIM = 2048
BINS = 1024
L = 16

_info = plsc.get_sparse_core_info()
NC, NS = _info.num_cores, _info.num_subcores
NW = NC * NS
ROWS_W = DIM // NW
CH_ROWS = 8
NCH = ROWS_W // CH_ROWS
VECS = CH_ROWS * DIM // L
UNROLL = 8
POSBIT = 0x40000000


def _make_sc_pass(shift, mask_shift):
    mesh = plsc.VectorSubcoreMesh(core_axis_name="c", subcore_axis_name="s")
    is_p1 = mask_shift >= 30
    in_dtype = jnp.float32 if is_p1 else jnp.int32
    out_type = [jax.ShapeDtypeStruct((NW * BINS,), jnp.float32)]
    scratch = [
        pltpu.VMEM((BINS * L,), jnp.float32),
        pltpu.VMEM((CH_ROWS, DIM), in_dtype),
        pltpu.VMEM((CH_ROWS, DIM), in_dtype),
        pltpu.VMEM((BINS,), jnp.float32),
        pltpu.VMEM((L,), jnp.int32),
        pltpu.SemaphoreType.DMA,
        pltpu.SemaphoreType.DMA,
    ]
    if is_p1:
        out_type.append(jax.ShapeDtypeStruct((DIM, DIM), jnp.int32))
        scratch += [
            pltpu.VMEM((CH_ROWS, DIM), jnp.float32),
            pltpu.VMEM((CH_ROWS, DIM), jnp.float32),
            pltpu.VMEM((CH_ROWS, DIM), jnp.int32),
            pltpu.VMEM((CH_ROWS, DIM), jnp.int32),
            pltpu.SemaphoreType.DMA,
            pltpu.SemaphoreType.DMA,
        ]

    def body(*args):
        if is_p1:
            (pred_hbm, prob_hbm, mval_hbm, hist_out, key_out,
             hist_v, pb0, pb1, merged, mvalbuf, sem0, sem1,
             mb0, mb1, kb0, kb1, osem0, osem1) = args
            mbufs, kbufs, osems = (mb0, mb1), (kb0, kb1), (osem0, osem1)
        else:
            (pred_hbm, mval_hbm, hist_out,
             hist_v, pb0, pb1, merged, mvalbuf, sem0, sem1) = args
            prob_hbm = key_out = None
        pbufs, sems = (pb0, pb1), (sem0, sem1)

        wid = lax.axis_index("s") * NC + lax.axis_index("c")
        row0 = wid * ROWS_W
        pltpu.sync_copy(mval_hbm, mvalbuf)
        mval = mvalbuf[...]
        lane = lax.iota(jnp.int32, L)
        laneB = lane * jnp.full((L,), BINS, jnp.int32)
        shv = jnp.full((L,), shift, jnp.int32)
        mshv = jnp.full((L,), mask_shift, jnp.int32)
        binmask = jnp.full((L,), BINS - 1, jnp.int32)
        posbit = jnp.full((L,), POSBIT, jnp.int32)
        z16f = jnp.zeros((L,), jnp.float32)
        one16f = jnp.full((L,), 1.0, jnp.float32)

        def zbody(j, c):
            hist_v[pl.ds(j * L, L)] = z16f
            return c

        lax.fori_loop(0, BINS * L // L, zbody, 0)

        def issue(b, ch):
            r = row0 + ch * CH_ROWS
            pltpu.async_copy(pred_hbm.at[pl.ds(r, CH_ROWS)], pbufs[b], sems[b])
            if is_p1:
                pltpu.async_copy(
                    prob_hbm.at[pl.ds(r, CH_ROWS)], mbufs[b], sems[b])

        def wait(b):
            src = pred_hbm.at[pl.ds(row0, CH_ROWS)]
            pltpu.make_async_copy(src, pbufs[b], sems[b]).wait()
            if is_p1:
                pltpu.make_async_copy(src, mbufs[b], sems[b]).wait()

        def process(b, ch, carry):
            wait(b)
            if is_p1:
                @pl.when(ch >= 2)
                def _():
                    pltpu.make_async_copy(
                        kbufs[b], key_out.at[pl.ds(row0, CH_ROWS)],
                        osems[b]).wait()
            pbuf = pbufs[b]

            def vec_body(j, c):
                for r in range(CH_ROWS):
                    ps, ms = [], []
                    for u in range(UNROLL):
                        sl = pl.ds(j * (UNROLL * L) + u * L, L)
                        ps.append(pbuf[r, sl])
                        if is_p1:
                            ms.append(mbufs[b][r, sl])
                    idxs, vals, masks, keys = [], [], [], []
                    for u in range(UNROLL):
                        bits = lax.bitcast_convert_type(ps[u], jnp.int32)
                        sh = lax.shift_right_logical(bits, shv) if shift else bits
                        bin_ = sh & binmask if shift != 20 else sh
                        idxs.append(laneB | bin_)
                        if is_p1:
                            msk = ms[u] == z16f
                            keys.append(jnp.where(msk, bits, bits | posbit))
                        else:
                            msk = lax.shift_right_logical(bits, mshv) == mval
                        masks.append(msk)
                        vals.append(one16f)
                    for u in range(UNROLL):
                        plsc.addupdate_scatter(
                            hist_v, [idxs[u]], vals[u], mask=masks[u])
                        if is_p1:
                            sl = pl.ds(j * (UNROLL * L) + u * L, L)
                            kbufs[b][r, sl] = keys[u]
                return c

            nj = DIM // (UNROLL * L)
            lax.fori_loop(0, nj, vec_body, carry)
            if is_p1:
                r = row0 + ch * CH_ROWS
                pltpu.async_copy(
                    kbufs[b], key_out.at[pl.ds(r, CH_ROWS)], osems[b])
            @pl.when(ch + 2 < NCH)
            def _():
                issue(b, ch + 2)
            return carry

        issue(0, 0)
        issue(1, 1)

        def pair(cp, c):
            c = process(0, cp * 2, c)
            c = process(1, cp * 2 + 1, c)
            return c

        lax.fori_loop(0, NCH // 2, pair, 0)
        if is_p1:
            for b in range(2):
                pltpu.make_async_copy(
                    kbufs[b], key_out.at[pl.ds(row0, CH_ROWS)],
                    osems[b]).wait()

        def mbody(j, c):
            acc = hist_v[pl.ds(j * L, L)]
            for l in range(1, L):
                acc = acc + hist_v[pl.ds(l * BINS + j * L, L)]
            merged[pl.ds(j * L, L)] = acc
            return c

        lax.fori_loop(0, BINS // L, mbody, 0)
        pltpu.sync_copy(merged, hist_out.at[pl.ds(wid * BINS, BINS)])

    return functools.partial(
        pl.kernel, mesh=mesh, out_type=out_type, scratch_types=scratch,
        compiler_params=pltpu.CompilerParams(
            needs_layout_passes=False, use_tc_tiling_on_sc=True),
    )(body)


_sc_pass1 = _make_sc_pass(20, 30)
_sc_pass2 = _make_sc_pass(10, 20)
_sc_pass3 = _make_sc_pass(0, 10)

TC_ROWS = DIM
TC_COLS = DIM
TC_BLK = 256


def _tc_body(t_ref, key_ref, pos_out, sel_out):
    i = pl.program_id(0)
    key = key_ref[...]
    pos = key >= POSBIT
    bits = key & (POSBIT - 1)
    p = lax.bitcast_convert_type(bits, jnp.float32)
    x = jnp.where(pos, p, 1.0 - p)
    nlogx = -jnp.maximum(jnp.log(x), -100.0)
    pos_part = jnp.sum(jnp.where(pos, nlogx, 0.0))
    sel_part = jnp.sum(jnp.where((~pos) & (bits > t_ref[0]), nlogx, 0.0))

    @pl.when(i == 0)
    def _():
        pos_out[...] = jnp.zeros((1, 1), jnp.float32)
        sel_out[...] = jnp.zeros((1, 1), jnp.float32)

    pos_out[...] += jnp.full((1, 1), pos_part, jnp.float32)
    sel_out[...] += jnp.full((1, 1), sel_part, jnp.float32)


def _tc_sums(t_bits, keys):
    return pl.pallas_call(
        _tc_body,
        grid=(TC_ROWS // TC_BLK,),
        in_specs=[
            pl.BlockSpec(memory_space=pltpu.SMEM),
            pl.BlockSpec((TC_BLK, TC_COLS), lambda i: (i, 0)),
        ],
        out_specs=[
            pl.BlockSpec((1, 1), lambda i: (0, 0)),
            pl.BlockSpec((1, 1), lambda i: (0, 0)),
        ],
        out_shape=[
            jax.ShapeDtypeStruct((1, 1), jnp.float32),
            jax.ShapeDtypeStruct((1, 1), jnp.float32),
        ],
    )(t_bits, keys)


def _select_bin(hist, want):
    frev = jnp.cumsum(hist[::-1])[::-1]
    b = jnp.sum((frev >= want).astype(jnp.int32)) - 1
    above = jnp.concatenate([frev, jnp.zeros((1,), jnp.float32)])[b + 1]
    return b, above, want - above


def kernel(pred, prob_map, prob_mask):
    predf = pred.reshape(DIM, DIM)
    probf = prob_map.reshape(DIM, DIM)

    z16 = jnp.zeros((L,), jnp.int32)
    hist1_rows, keys = _sc_pass1(predf, probf, z16)
    hist1 = hist1_rows.reshape(NW, BINS).sum(axis=0)
    neg_count = hist1.sum()
    pos_num = jnp.float32(N) - neg_count
    neg_num = jnp.minimum(neg_count, jnp.floor(pos_num * 3.0))

    b1, above1, r1 = _select_bin(hist1, neg_num)

    mv2 = jnp.full((L,), b1, jnp.int32)
    (hist2_rows,) = _sc_pass2(keys, mv2)
    hist2 = hist2_rows.reshape(NW, BINS).sum(axis=0)
    b2, above2, r2 = _select_bin(hist2, r1)

    mv3 = jnp.full((L,), b1 * BINS + b2, jnp.int32)
    (hist3_rows,) = _sc_pass3(keys, mv3)
    hist3 = hist3_rows.reshape(NW, BINS).sum(axis=0)
    b3, above3, r_final = _select_bin(hist3, r2)

    t_bits = (b1 * (1 << 20) + b2 * (1 << 10) + b3).astype(jnp.int32)

    pos_sum, sel_sum = _tc_sums(t_bits.reshape(1), keys)

    t_f = lax.bitcast_convert_type(t_bits, jnp.float32)
    loss_t = -jnp.maximum(jnp.log(1.0 - t_f), -100.0)
    tie_sum = jnp.where(r_final > 0.0, r_final * loss_t, 0.0)

    bce = (pos_sum[0, 0] + sel_sum[0, 0] + tie_sum) / (pos_num + neg_num + 1e-6)
    return bce
